# Initial kernel scaffold; baseline (speedup 1.0000x reference)
import jax
import jax.numpy as jnp
from jax.experimental import pallas as pl


def kernel(x, edge_index_r0, edge_index_r1, edge_index_r2, W0_r0, b0_r0, W0_r1, b0_r1, W0_r2, b0_r2, W1_r0, b1_r0, W1_r1, b1_r1, W1_r2, b1_r2):
    raise NotImplementedError("write your pallas kernel here")


# speedup vs baseline: 1.8952x; 1.8952x over previous
import functools

import jax
import jax.numpy as jnp
from jax import lax
from jax.experimental import pallas as pl
from jax.experimental.pallas import tpu as pltpu
from jax.experimental.pallas import tpu_sc as plsc

N = 50000
D = 128
E = 200000

NPAD = 51200
EPAD = 200704
CHUNKS = 98
K = 128
BATCH = 14
NBATCH = CHUNKS // BATCH
ROWS_PER_TILE = NPAD // 16
ZCH = ROWS_PER_TILE // K
NP_PASS = 4
DP = D // NP_PASS

_mesh = plsc.VectorSubcoreMesh(core_axis_name="c", subcore_axis_name="s")


@functools.partial(
    pl.kernel,
    out_type=jax.ShapeDtypeStruct((3, NP_PASS, NPAD, DP), jnp.float32),
    mesh=_mesh,
    scratch_types=[
        pltpu.VMEM((BATCH, K), jnp.int32),
        pltpu.VMEM((BATCH, K), jnp.int32),
        pltpu.VMEM((K, DP), jnp.float32),
        pltpu.VMEM((K, DP), jnp.float32),
        pltpu.VMEM_SHARED((NPAD, DP), jnp.float32),
        pltpu.SemaphoreType.DMA,
    ],
    compiler_params=pltpu.CompilerParams(use_tc_tiling_on_sc=False),
)
def _sc_agg(hp_hbm, src_hbm, dst_hbm, z_hbm, out_hbm,
            src_v, dst_v, stage_v, zero_v, acc, sem):
    c = lax.axis_index("c")
    s = lax.axis_index("s")
    pltpu.sync_copy(z_hbm, zero_v)
    for r in range(3):
        for pp in range(2):
            p = 2 * c + pp
            @pl.loop(0, ZCH)
            def _zero(j):
                pltpu.sync_copy(
                    zero_v, acc.at[pl.ds(s * ROWS_PER_TILE + j * K, K)])
            plsc.subcore_barrier()

            @pl.loop(0, NBATCH)
            def _batch(b):
                pltpu.sync_copy(src_hbm.at[r, s, pl.ds(b * BATCH, BATCH)],
                                src_v)
                pltpu.sync_copy(dst_hbm.at[r, s, pl.ds(b * BATCH, BATCH)],
                                dst_v)

                @pl.loop(0, BATCH)
                def _edges(j):
                    pltpu.async_copy(
                        hp_hbm.at[p].at[src_v.at[j]], stage_v, sem).wait()
                    pltpu.sync_copy(stage_v, acc.at[dst_v.at[j]], add=True)
            plsc.subcore_barrier()

            pltpu.sync_copy(
                acc.at[pl.ds(s * ROWS_PER_TILE, ROWS_PER_TILE)],
                out_hbm.at[r, p, pl.ds(s * ROWS_PER_TILE, ROWS_PER_TILE)])


@functools.partial(
    pl.kernel,
    out_type=jax.ShapeDtypeStruct((3, 2, NPAD, 16), jnp.float32),
    mesh=_mesh,
    scratch_types=[
        pltpu.VMEM((CHUNKS, K), jnp.int32),
        pltpu.VMEM((K, 16), jnp.float32),
        pltpu.VMEM((K, 16), jnp.float32),
        pltpu.VMEM_SHARED((NPAD, 16), jnp.float32),
    ],
    compiler_params=pltpu.CompilerParams(use_tc_tiling_on_sc=False),
)
def _sc_counts(dst_hbm, ones_hbm, z_hbm, out_hbm,
               dst_v, ones_v, zero_v, acc):
    c = lax.axis_index("c")
    s = lax.axis_index("s")
    pltpu.sync_copy(ones_hbm, ones_v)
    pltpu.sync_copy(z_hbm, zero_v)
    for r in range(3):
        pltpu.sync_copy(dst_hbm.at[r, s], dst_v)

        @pl.loop(0, ZCH)
        def _zero(j):
            pltpu.sync_copy(
                zero_v, acc.at[pl.ds(s * ROWS_PER_TILE + j * K, K)])
        plsc.subcore_barrier()

        @pl.loop(0, CHUNKS)
        def _edges(j):
            pltpu.sync_copy(ones_v, acc.at[dst_v.at[j]], add=True)
        plsc.subcore_barrier()

        pltpu.sync_copy(
            acc.at[pl.ds(s * ROWS_PER_TILE, ROWS_PER_TILE)],
            out_hbm.at[r, c, pl.ds(s * ROWS_PER_TILE, ROWS_PER_TILE)])


def _tc_body(parts_ref, cnt_ref, w_ref, b_ref, out_ref, *, emit_parts):
    acc = jnp.zeros((K, D), jnp.float32)
    for r in range(3):
        cnt = cnt_ref[r, 0, :, 0:1]
        inv = 1.0 / jnp.maximum(cnt, 1.0)
        s = jnp.concatenate([parts_ref[r, p] for p in range(NP_PASS)], axis=1)
        agg = s * inv
        acc = acc + jnp.dot(agg, w_ref[r],
                            preferred_element_type=jnp.float32)
        acc = acc + jnp.where(cnt > 0.0, 1.0, 0.0) * b_ref[r]
    y = jnp.where(acc >= 0.0, acc, 0.01 * acc)
    if emit_parts:
        for p in range(NP_PASS):
            out_ref[p] = y[:, p * DP:(p + 1) * DP]
    else:
        out_ref[...] = y


def _tc_mm(parts, cnts, wstack, bstack, emit_parts):
    grid = (NPAD // K,)
    if emit_parts:
        out_shape = jax.ShapeDtypeStruct((NP_PASS, NPAD, DP), jnp.float32)
        out_spec = pl.BlockSpec((NP_PASS, K, DP), lambda i: (0, i, 0))
    else:
        out_shape = jax.ShapeDtypeStruct((NPAD, D), jnp.float32)
        out_spec = pl.BlockSpec((K, D), lambda i: (i, 0))
    return pl.pallas_call(
        functools.partial(_tc_body, emit_parts=emit_parts),
        grid=grid,
        in_specs=[
            pl.BlockSpec((3, NP_PASS, K, DP), lambda i: (0, 0, i, 0)),
            pl.BlockSpec((3, 1, K, 16), lambda i: (0, 0, i, 0)),
            pl.BlockSpec((3, D, D), lambda i: (0, 0, 0)),
            pl.BlockSpec((3, 1, D), lambda i: (0, 0, 0)),
        ],
        out_specs=out_spec,
        out_shape=out_shape,
    )(parts, cnts, wstack, bstack)


def _prep_edges(ei):
    src = jnp.concatenate(
        [ei[0].astype(jnp.int32), jnp.zeros((EPAD - E,), jnp.int32)])
    dst = jnp.concatenate(
        [ei[1].astype(jnp.int32), jnp.full((EPAD - E,), N, jnp.int32)])
    return src, dst


def kernel(x, edge_index_r0, edge_index_r1, edge_index_r2,
           W0_r0, b0_r0, W0_r1, b0_r1, W0_r2, b0_r2,
           W1_r0, b1_r0, W1_r1, b1_r1, W1_r2, b1_r2):
    prepped = [_prep_edges(e)
               for e in (edge_index_r0, edge_index_r1, edge_index_r2)]
    src3 = jnp.stack([p[0] for p in prepped]).reshape(3, 16, CHUNKS, K)
    dst3 = jnp.stack([p[1] for p in prepped]).reshape(3, 16, CHUNKS, K)

    z32 = jnp.zeros((K, DP), jnp.float32)
    z16 = jnp.zeros((K, 16), jnp.float32)
    o16 = jnp.ones((K, 16), jnp.float32)

    xp = jnp.pad(x, ((0, NPAD - N), (0, 0)))
    hparts0 = xp.reshape(NPAD, NP_PASS, DP).transpose(1, 0, 2)

    w0 = jnp.stack([W0_r0, W0_r1, W0_r2])
    b0 = jnp.stack([b0_r0, b0_r1, b0_r2])[:, None, :]
    w1 = jnp.stack([W1_r0, W1_r1, W1_r2])
    b1 = jnp.stack([b1_r0, b1_r1, b1_r2])[:, None, :]

    cnts = _sc_counts(dst3, o16, z16)
    parts0 = _sc_agg(hparts0, src3, dst3, z32)
    hparts1 = _tc_mm(parts0, cnts, w0, b0, emit_parts=True)
    parts1 = _sc_agg(hparts1, src3, dst3, z32)
    out = _tc_mm(parts1, cnts, w1, b1, emit_parts=False)
    return out[:N]


# speedup vs baseline: 2.4822x; 1.3097x over previous
import functools

import jax
import jax.numpy as jnp
from jax import lax
from jax.experimental import pallas as pl
from jax.experimental.pallas import tpu as pltpu
from jax.experimental.pallas import tpu_sc as plsc

N = 50000
D = 128
E = 200000

NPAD = 51200
EPAD = 200704
CHUNKS = 98
K = 128
BATCH = 14
NBATCH = CHUNKS // BATCH
ROWS_PER_TILE = NPAD // 16
ZCH = ROWS_PER_TILE // K
NP_PASS = 4
DP = D // NP_PASS

_mesh = plsc.VectorSubcoreMesh(core_axis_name="c", subcore_axis_name="s")


@functools.partial(
    pl.kernel,
    out_type=jax.ShapeDtypeStruct((3, NPAD, D), jnp.float32),
    mesh=_mesh,
    scratch_types=[
        pltpu.VMEM((BATCH, K), jnp.int32),
        pltpu.VMEM((BATCH, K), jnp.int32),
        pltpu.VMEM((K, DP), jnp.float32),
        pltpu.VMEM((K, DP), jnp.float32),
        pltpu.VMEM((K, DP), jnp.float32),
        pltpu.VMEM_SHARED((NPAD, DP), jnp.float32),
        pltpu.SemaphoreType.DMA,
        pltpu.SemaphoreType.DMA,
    ],
    compiler_params=pltpu.CompilerParams(use_tc_tiling_on_sc=False),
)
def _sc_agg(hp_hbm, src_hbm, dst_hbm, z_hbm, out_hbm,
            src_v, dst_v, stage_a, stage_b, zero_v, acc, sem_a, sem_b):
    c = lax.axis_index("c")
    s = lax.axis_index("s")
    pltpu.sync_copy(z_hbm, zero_v)
    for r in range(3):
        for pp in range(2):
            p = 2 * c + pp
            @pl.loop(0, ZCH)
            def _zero(j):
                pltpu.sync_copy(
                    zero_v, acc.at[pl.ds(s * ROWS_PER_TILE + j * K, K)])
            plsc.subcore_barrier()

            hp_p = hp_hbm.at[p]

            @pl.loop(0, NBATCH)
            def _batch(b):
                pltpu.sync_copy(src_hbm.at[r, s, pl.ds(b * BATCH, BATCH)],
                                src_v)
                pltpu.sync_copy(dst_hbm.at[r, s, pl.ds(b * BATCH, BATCH)],
                                dst_v)
                pltpu.async_copy(hp_p.at[src_v.at[0]], stage_a, sem_a)

                @pl.loop(0, BATCH // 2)
                def _pair(t):
                    pltpu.make_async_copy(
                        hp_p.at[src_v.at[2 * t]], stage_a, sem_a).wait()
                    pltpu.async_copy(
                        hp_p.at[src_v.at[2 * t + 1]], stage_b, sem_b)
                    pltpu.sync_copy(stage_a, acc.at[dst_v.at[2 * t]],
                                    add=True)
                    pltpu.make_async_copy(
                        hp_p.at[src_v.at[2 * t + 1]], stage_b, sem_b).wait()

                    @pl.when(t < BATCH // 2 - 1)
                    def _issue_next():
                        pltpu.async_copy(
                            hp_p.at[src_v.at[2 * t + 2]], stage_a, sem_a)
                    pltpu.sync_copy(stage_b, acc.at[dst_v.at[2 * t + 1]],
                                    add=True)
            plsc.subcore_barrier()

            pltpu.sync_copy(
                acc.at[pl.ds(s * ROWS_PER_TILE, ROWS_PER_TILE)],
                out_hbm.at[r,
                           pl.ds(s * ROWS_PER_TILE, ROWS_PER_TILE),
                           pl.ds(p * DP, DP)])


@functools.partial(
    pl.kernel,
    out_type=jax.ShapeDtypeStruct((3, 2, NPAD, 16), jnp.float32),
    mesh=_mesh,
    scratch_types=[
        pltpu.VMEM((CHUNKS, K), jnp.int32),
        pltpu.VMEM((K, 16), jnp.float32),
        pltpu.VMEM((K, 16), jnp.float32),
        pltpu.VMEM_SHARED((NPAD, 16), jnp.float32),
    ],
    compiler_params=pltpu.CompilerParams(use_tc_tiling_on_sc=False),
)
def _sc_counts(dst_hbm, ones_hbm, z_hbm, out_hbm,
               dst_v, ones_v, zero_v, acc):
    c = lax.axis_index("c")
    s = lax.axis_index("s")
    pltpu.sync_copy(ones_hbm, ones_v)
    pltpu.sync_copy(z_hbm, zero_v)
    for r in range(3):
        pltpu.sync_copy(dst_hbm.at[r, s], dst_v)

        @pl.loop(0, ZCH)
        def _zero(j):
            pltpu.sync_copy(
                zero_v, acc.at[pl.ds(s * ROWS_PER_TILE + j * K, K)])
        plsc.subcore_barrier()

        @pl.loop(0, CHUNKS)
        def _edges(j):
            pltpu.sync_copy(ones_v, acc.at[dst_v.at[j]], add=True)
        plsc.subcore_barrier()

        pltpu.sync_copy(
            acc.at[pl.ds(s * ROWS_PER_TILE, ROWS_PER_TILE)],
            out_hbm.at[r, c, pl.ds(s * ROWS_PER_TILE, ROWS_PER_TILE)])


def _tc_body(parts_ref, cnt_ref, w_ref, b_ref, out_ref, *, emit_parts):
    acc = jnp.zeros((K, D), jnp.float32)
    for r in range(3):
        cnt = cnt_ref[r, 0, :, 0:1]
        inv = 1.0 / jnp.maximum(cnt, 1.0)
        agg = parts_ref[r] * inv
        acc = acc + jnp.dot(agg, w_ref[r],
                            preferred_element_type=jnp.float32)
        acc = acc + jnp.where(cnt > 0.0, 1.0, 0.0) * b_ref[r]
    y = jnp.where(acc >= 0.0, acc, 0.01 * acc)
    if emit_parts:
        for p in range(NP_PASS):
            out_ref[p] = y[:, p * DP:(p + 1) * DP]
    else:
        out_ref[...] = y


def _tc_mm(parts, cnts, wstack, bstack, emit_parts):
    grid = (NPAD // K,)
    if emit_parts:
        out_shape = jax.ShapeDtypeStruct((NP_PASS, NPAD, DP), jnp.float32)
        out_spec = pl.BlockSpec((NP_PASS, K, DP), lambda i: (0, i, 0))
    else:
        out_shape = jax.ShapeDtypeStruct((NPAD, D), jnp.float32)
        out_spec = pl.BlockSpec((K, D), lambda i: (i, 0))
    return pl.pallas_call(
        functools.partial(_tc_body, emit_parts=emit_parts),
        grid=grid,
        in_specs=[
            pl.BlockSpec((3, K, D), lambda i: (0, i, 0)),
            pl.BlockSpec((3, 1, K, 16), lambda i: (0, 0, i, 0)),
            pl.BlockSpec((3, D, D), lambda i: (0, 0, 0)),
            pl.BlockSpec((3, 1, D), lambda i: (0, 0, 0)),
        ],
        out_specs=out_spec,
        out_shape=out_shape,
    )(parts, cnts, wstack, bstack)


def _prep_edges(ei):
    src = jnp.concatenate(
        [ei[0].astype(jnp.int32), jnp.zeros((EPAD - E,), jnp.int32)])
    dst = jnp.concatenate(
        [ei[1].astype(jnp.int32), jnp.full((EPAD - E,), N, jnp.int32)])
    return src, dst


def kernel(x, edge_index_r0, edge_index_r1, edge_index_r2,
           W0_r0, b0_r0, W0_r1, b0_r1, W0_r2, b0_r2,
           W1_r0, b1_r0, W1_r1, b1_r1, W1_r2, b1_r2):
    prepped = [_prep_edges(e)
               for e in (edge_index_r0, edge_index_r1, edge_index_r2)]
    src3 = jnp.stack([p[0] for p in prepped]).reshape(3, 16, CHUNKS, K)
    dst3 = jnp.stack([p[1] for p in prepped]).reshape(3, 16, CHUNKS, K)

    z32 = jnp.zeros((K, DP), jnp.float32)
    z16 = jnp.zeros((K, 16), jnp.float32)
    o16 = jnp.ones((K, 16), jnp.float32)

    xp = jnp.pad(x, ((0, NPAD - N), (0, 0)))
    hparts0 = xp.reshape(NPAD, NP_PASS, DP).transpose(1, 0, 2)

    w0 = jnp.stack([W0_r0, W0_r1, W0_r2])
    b0 = jnp.stack([b0_r0, b0_r1, b0_r2])[:, None, :]
    w1 = jnp.stack([W1_r0, W1_r1, W1_r2])
    b1 = jnp.stack([b1_r0, b1_r1, b1_r2])[:, None, :]

    cnts = _sc_counts(dst3, o16, z16)
    parts0 = _sc_agg(hparts0, src3, dst3, z32)
    hparts1 = _tc_mm(parts0, cnts, w0, b0, emit_parts=True)
    parts1 = _sc_agg(hparts1, src3, dst3, z32)
    out = _tc_mm(parts1, cnts, w1, b1, emit_parts=False)
    return out[:N]


# speedup vs baseline: 3.2752x; 1.3195x over previous
import functools

import jax
import jax.numpy as jnp
from jax import lax
from jax.experimental import pallas as pl
from jax.experimental.pallas import tpu as pltpu
from jax.experimental.pallas import tpu_sc as plsc

N = 50000
D = 128
E = 200000

NPAD = 51200
EPAD = 200704
CHUNKS = 98
K = 128
BATCH = 14
NBATCH = CHUNKS // BATCH
BK = 512
ROWS_PER_TILE = NPAD // 16
ZCH = ROWS_PER_TILE // K
NP_PASS = 4
DP = D // NP_PASS

_mesh = plsc.VectorSubcoreMesh(core_axis_name="c", subcore_axis_name="s")


@functools.partial(
    pl.kernel,
    out_type=jax.ShapeDtypeStruct((3, NPAD, D), jnp.float32),
    mesh=_mesh,
    scratch_types=[
        pltpu.VMEM((BATCH, K), jnp.int32),
        pltpu.VMEM((BATCH, K), jnp.int32),
        pltpu.VMEM((K, DP), jnp.float32),
        pltpu.VMEM((K, DP), jnp.float32),
        pltpu.VMEM((K, DP), jnp.float32),
        pltpu.VMEM((K, DP), jnp.float32),
        pltpu.VMEM_SHARED((NPAD, DP), jnp.float32),
        pltpu.SemaphoreType.DMA,
        pltpu.SemaphoreType.DMA,
        pltpu.SemaphoreType.DMA,
        pltpu.SemaphoreType.DMA,
        pltpu.SemaphoreType.DMA,
        pltpu.SemaphoreType.DMA,
        pltpu.SemaphoreType.DMA,
        pltpu.SemaphoreType.DMA,
        pltpu.SemaphoreType.DMA,
    ],
    compiler_params=pltpu.CompilerParams(use_tc_tiling_on_sc=False),
)
def _sc_agg(hp_hbm, src_hbm, dst_hbm, z_hbm, out_hbm,
            src_v, dst_v, st0, st1, st2, st3, acc,
            sg0, sg1, sg2, sg3, ss0, ss1, ss2, ss3, sz):
    c = lax.axis_index("c")
    s = lax.axis_index("s")
    stg = [st0, st1, st2, st3]
    sgs = [sg0, sg1, sg2, sg3]
    sss = [ss0, ss1, ss2, ss3]
    for r in range(3):
        for pp in range(2):
            p = 2 * c + pp
            @pl.loop(0, ZCH)
            def _zero(j):
                pltpu.async_copy(
                    z_hbm, acc.at[pl.ds(s * ROWS_PER_TILE + j * K, K)], sz)

            @pl.loop(0, ZCH)
            def _zero_wait(j):
                pltpu.make_async_copy(
                    z_hbm, acc.at[pl.ds(s * ROWS_PER_TILE + j * K, K)],
                    sz).wait()
            plsc.subcore_barrier()

            hp_p = hp_hbm.at[p]

            @pl.loop(0, NBATCH)
            def _batch(b):
                pltpu.sync_copy(src_hbm.at[r, s, pl.ds(b * BATCH, BATCH)],
                                src_v)
                pltpu.sync_copy(dst_hbm.at[r, s, pl.ds(b * BATCH, BATCH)],
                                dst_v)
                for q in range(4):
                    pltpu.async_copy(hp_p.at[src_v.at[q]], stg[q], sgs[q])
                for t in range(BATCH):
                    k = t % 4
                    pltpu.make_async_copy(
                        hp_p.at[src_v.at[t]], stg[k], sgs[k]).wait()
                    pltpu.async_copy(
                        stg[k], acc.at[dst_v.at[t]], sss[k], add=True)
                    if t + 4 < BATCH:
                        pltpu.make_async_copy(
                            stg[k], acc.at[dst_v.at[t]], sss[k]).wait()
                        pltpu.async_copy(
                            hp_p.at[src_v.at[t + 4]], stg[k], sgs[k])
                for t in range(max(0, BATCH - 4), BATCH):
                    k = t % 4
                    pltpu.make_async_copy(
                        stg[k], acc.at[dst_v.at[t]], sss[k]).wait()
            plsc.subcore_barrier()

            pltpu.sync_copy(
                acc.at[pl.ds(s * ROWS_PER_TILE, ROWS_PER_TILE)],
                out_hbm.at[r,
                           pl.ds(s * ROWS_PER_TILE, ROWS_PER_TILE),
                           pl.ds(p * DP, DP)])


@functools.partial(
    pl.kernel,
    out_type=jax.ShapeDtypeStruct((3, 2, NPAD, 16), jnp.float32),
    mesh=_mesh,
    scratch_types=[
        pltpu.VMEM((CHUNKS, K), jnp.int32),
        pltpu.VMEM((K, 16), jnp.float32),
        pltpu.VMEM((K, 16), jnp.float32),
        pltpu.VMEM_SHARED((NPAD, 16), jnp.float32),
        pltpu.SemaphoreType.DMA,
    ],
    compiler_params=pltpu.CompilerParams(use_tc_tiling_on_sc=False),
)
def _sc_counts(dst_hbm, ones_hbm, z_hbm, out_hbm,
               dst_v, ones_v, zero_v, acc, sem):
    c = lax.axis_index("c")
    s = lax.axis_index("s")
    pltpu.sync_copy(ones_hbm, ones_v)
    pltpu.sync_copy(z_hbm, zero_v)
    for r in range(3):
        pltpu.sync_copy(dst_hbm.at[r, s], dst_v)

        @pl.loop(0, ZCH)
        def _zero(j):
            pltpu.sync_copy(
                zero_v, acc.at[pl.ds(s * ROWS_PER_TILE + j * K, K)])
        plsc.subcore_barrier()

        @pl.loop(0, CHUNKS)
        def _edges(j):
            pltpu.async_copy(ones_v, acc.at[dst_v.at[j]], sem, add=True)

        @pl.loop(0, CHUNKS)
        def _edges_wait(j):
            pltpu.make_async_copy(
                ones_v, acc.at[dst_v.at[j]], sem).wait()
        plsc.subcore_barrier()

        pltpu.sync_copy(
            acc.at[pl.ds(s * ROWS_PER_TILE, ROWS_PER_TILE)],
            out_hbm.at[r, c, pl.ds(s * ROWS_PER_TILE, ROWS_PER_TILE)])


def _tc_body(parts_ref, cnt_ref, w_ref, b_ref, out_ref, *, emit_parts):
    acc = jnp.zeros((BK, D), jnp.float32)
    for r in range(3):
        cnt = cnt_ref[r, 0, :, 0:1]
        inv = 1.0 / jnp.maximum(cnt, 1.0)
        agg = parts_ref[r] * inv
        acc = acc + jnp.dot(agg, w_ref[r],
                            preferred_element_type=jnp.float32)
        acc = acc + jnp.where(cnt > 0.0, 1.0, 0.0) * b_ref[r]
    y = jnp.where(acc >= 0.0, acc, 0.01 * acc)
    if emit_parts:
        for p in range(NP_PASS):
            out_ref[p] = y[:, p * DP:(p + 1) * DP]
    else:
        out_ref[...] = y


def _tc_mm(parts, cnts, wstack, bstack, emit_parts):
    grid = (NPAD // BK,)
    if emit_parts:
        out_shape = jax.ShapeDtypeStruct((NP_PASS, NPAD, DP), jnp.float32)
        out_spec = pl.BlockSpec((NP_PASS, BK, DP), lambda i: (0, i, 0))
    else:
        out_shape = jax.ShapeDtypeStruct((NPAD, D), jnp.float32)
        out_spec = pl.BlockSpec((BK, D), lambda i: (i, 0))
    return pl.pallas_call(
        functools.partial(_tc_body, emit_parts=emit_parts),
        grid=grid,
        in_specs=[
            pl.BlockSpec((3, BK, D), lambda i: (0, i, 0)),
            pl.BlockSpec((3, 1, BK, 16), lambda i: (0, 0, i, 0)),
            pl.BlockSpec((3, D, D), lambda i: (0, 0, 0)),
            pl.BlockSpec((3, 1, D), lambda i: (0, 0, 0)),
        ],
        out_specs=out_spec,
        out_shape=out_shape,
    )(parts, cnts, wstack, bstack)


def _prep_edges(ei):
    src = jnp.concatenate(
        [ei[0].astype(jnp.int32), jnp.zeros((EPAD - E,), jnp.int32)])
    dst = jnp.concatenate(
        [ei[1].astype(jnp.int32), jnp.full((EPAD - E,), N, jnp.int32)])
    return src, dst


def kernel(x, edge_index_r0, edge_index_r1, edge_index_r2,
           W0_r0, b0_r0, W0_r1, b0_r1, W0_r2, b0_r2,
           W1_r0, b1_r0, W1_r1, b1_r1, W1_r2, b1_r2):
    prepped = [_prep_edges(e)
               for e in (edge_index_r0, edge_index_r1, edge_index_r2)]
    src3 = jnp.stack([p[0] for p in prepped]).reshape(3, 16, CHUNKS, K)
    dst3 = jnp.stack([p[1] for p in prepped]).reshape(3, 16, CHUNKS, K)

    z32 = jnp.zeros((K, DP), jnp.float32)
    z16 = jnp.zeros((K, 16), jnp.float32)
    o16 = jnp.ones((K, 16), jnp.float32)

    xp = jnp.pad(x, ((0, NPAD - N), (0, 0)))
    hparts0 = xp.reshape(NPAD, NP_PASS, DP).transpose(1, 0, 2)

    w0 = jnp.stack([W0_r0, W0_r1, W0_r2])
    b0 = jnp.stack([b0_r0, b0_r1, b0_r2])[:, None, :]
    w1 = jnp.stack([W1_r0, W1_r1, W1_r2])
    b1 = jnp.stack([b1_r0, b1_r1, b1_r2])[:, None, :]

    cnts = _sc_counts(dst3, o16, z16)
    parts0 = _sc_agg(hparts0, src3, dst3, z32)
    hparts1 = _tc_mm(parts0, cnts, w0, b0, emit_parts=True)
    parts1 = _sc_agg(hparts1, src3, dst3, z32)
    out = _tc_mm(parts1, cnts, w1, b1, emit_parts=False)
    return out[:N]


# speedup vs baseline: 3.8643x; 1.1799x over previous
import functools

import jax
import jax.numpy as jnp
from jax import lax
from jax.experimental import pallas as pl
from jax.experimental.pallas import tpu as pltpu
from jax.experimental.pallas import tpu_sc as plsc

N = 50000
D = 128
E = 200000

NPAD = 51200
EPAD = 200704
CHUNKS = 98
K = 128
BATCH = 14
NBATCH = CHUNKS // BATCH
RING = 5
BK = 512
ROWS_PER_TILE = NPAD // 16
ZCH = ROWS_PER_TILE // K
NP_PASS = 2
DP = D // NP_PASS

_mesh = plsc.VectorSubcoreMesh(core_axis_name="c", subcore_axis_name="s")


@functools.partial(
    pl.kernel,
    out_type=jax.ShapeDtypeStruct((3, NPAD, D), jnp.bfloat16),
    mesh=_mesh,
    scratch_types=(
        [pltpu.VMEM((BATCH, K), jnp.int32),
         pltpu.VMEM((BATCH, K), jnp.int32)]
        + [pltpu.VMEM((K, DP), jnp.bfloat16) for _ in range(RING)]
        + [pltpu.VMEM_SHARED((NPAD, DP), jnp.bfloat16)]
        + [pltpu.SemaphoreType.DMA] * (2 * RING + 1)
    ),
    compiler_params=pltpu.CompilerParams(use_tc_tiling_on_sc=False),
)
def _sc_agg(hp_hbm, src_hbm, dst_hbm, z_hbm, out_hbm, src_v, dst_v, *rest):
    stg = list(rest[:RING])
    acc = rest[RING]
    sgs = list(rest[RING + 1:2 * RING + 1])
    sss = list(rest[2 * RING + 1:3 * RING + 1])
    sz = rest[3 * RING + 1]
    c = lax.axis_index("c")
    s = lax.axis_index("s")
    hp_p = hp_hbm.at[c]
    for r in range(3):
        @pl.loop(0, ZCH)
        def _zero(j):
            pltpu.async_copy(
                z_hbm, acc.at[pl.ds(s * ROWS_PER_TILE + j * K, K)], sz)

        @pl.loop(0, ZCH)
        def _zero_wait(j):
            pltpu.make_async_copy(
                z_hbm, acc.at[pl.ds(s * ROWS_PER_TILE + j * K, K)],
                sz).wait()
        plsc.subcore_barrier()

        @pl.loop(0, NBATCH)
        def _batch(b):
            pltpu.sync_copy(src_hbm.at[r, s, pl.ds(b * BATCH, BATCH)], src_v)
            pltpu.sync_copy(dst_hbm.at[r, s, pl.ds(b * BATCH, BATCH)], dst_v)
            for q in range(RING):
                pltpu.async_copy(hp_p.at[src_v.at[q]], stg[q], sgs[q])
            for t in range(BATCH):
                k = t % RING
                if t >= 1 and (t - 1) + RING < BATCH:
                    k2 = (t - 1) % RING
                    pltpu.make_async_copy(
                        stg[k2], acc.at[dst_v.at[t - 1]], sss[k2]).wait()
                    pltpu.async_copy(
                        hp_p.at[src_v.at[t - 1 + RING]], stg[k2], sgs[k2])
                pltpu.make_async_copy(
                    hp_p.at[src_v.at[t]], stg[k], sgs[k]).wait()
                pltpu.async_copy(
                    stg[k], acc.at[dst_v.at[t]], sss[k], add=True)
            for t in range(BATCH - RING, BATCH):
                k = t % RING
                pltpu.make_async_copy(
                    stg[k], acc.at[dst_v.at[t]], sss[k]).wait()
        plsc.subcore_barrier()

        pltpu.sync_copy(
            acc.at[pl.ds(s * ROWS_PER_TILE, ROWS_PER_TILE)],
            out_hbm.at[r,
                       pl.ds(s * ROWS_PER_TILE, ROWS_PER_TILE),
                       pl.ds(c * DP, DP)])


@functools.partial(
    pl.kernel,
    out_type=jax.ShapeDtypeStruct((3, 2, NPAD, 16), jnp.float32),
    mesh=_mesh,
    scratch_types=[
        pltpu.VMEM((CHUNKS, K), jnp.int32),
        pltpu.VMEM((K, 16), jnp.float32),
        pltpu.VMEM((K, 16), jnp.float32),
        pltpu.VMEM_SHARED((NPAD, 16), jnp.float32),
        pltpu.SemaphoreType.DMA,
    ],
    compiler_params=pltpu.CompilerParams(use_tc_tiling_on_sc=False),
)
def _sc_counts(dst_hbm, ones_hbm, z_hbm, out_hbm,
               dst_v, ones_v, zero_v, acc, sem):
    c = lax.axis_index("c")
    s = lax.axis_index("s")
    pltpu.sync_copy(ones_hbm, ones_v)
    pltpu.sync_copy(z_hbm, zero_v)
    for r in range(3):
        pltpu.sync_copy(dst_hbm.at[r, s], dst_v)

        @pl.loop(0, ZCH)
        def _zero(j):
            pltpu.sync_copy(
                zero_v, acc.at[pl.ds(s * ROWS_PER_TILE + j * K, K)])
        plsc.subcore_barrier()

        @pl.loop(0, CHUNKS)
        def _edges(j):
            pltpu.async_copy(ones_v, acc.at[dst_v.at[j]], sem, add=True)

        @pl.loop(0, CHUNKS)
        def _edges_wait(j):
            pltpu.make_async_copy(
                ones_v, acc.at[dst_v.at[j]], sem).wait()
        plsc.subcore_barrier()

        pltpu.sync_copy(
            acc.at[pl.ds(s * ROWS_PER_TILE, ROWS_PER_TILE)],
            out_hbm.at[r, c, pl.ds(s * ROWS_PER_TILE, ROWS_PER_TILE)])


def _tc_body(parts_ref, cnt_ref, w_ref, b_ref, out_ref, *, emit_parts):
    acc = jnp.zeros((BK, D), jnp.float32)
    for r in range(3):
        cnt = cnt_ref[r, 0, :, 0:1]
        inv = 1.0 / jnp.maximum(cnt, 1.0)
        agg = parts_ref[r].astype(jnp.float32) * inv
        acc = acc + jnp.dot(agg, w_ref[r],
                            preferred_element_type=jnp.float32)
        acc = acc + jnp.where(cnt > 0.0, 1.0, 0.0) * b_ref[r]
    y = jnp.where(acc >= 0.0, acc, 0.01 * acc)
    if emit_parts:
        for p in range(NP_PASS):
            out_ref[p] = y[:, p * DP:(p + 1) * DP].astype(jnp.bfloat16)
    else:
        out_ref[...] = y


def _tc_mm(parts, cnts, wstack, bstack, emit_parts):
    grid = (NPAD // BK,)
    if emit_parts:
        out_shape = jax.ShapeDtypeStruct((NP_PASS, NPAD, DP), jnp.bfloat16)
        out_spec = pl.BlockSpec((NP_PASS, BK, DP), lambda i: (0, i, 0))
    else:
        out_shape = jax.ShapeDtypeStruct((NPAD, D), jnp.float32)
        out_spec = pl.BlockSpec((BK, D), lambda i: (i, 0))
    return pl.pallas_call(
        functools.partial(_tc_body, emit_parts=emit_parts),
        grid=grid,
        in_specs=[
            pl.BlockSpec((3, BK, D), lambda i: (0, i, 0)),
            pl.BlockSpec((3, 1, BK, 16), lambda i: (0, 0, i, 0)),
            pl.BlockSpec((3, D, D), lambda i: (0, 0, 0)),
            pl.BlockSpec((3, 1, D), lambda i: (0, 0, 0)),
        ],
        out_specs=out_spec,
        out_shape=out_shape,
    )(parts, cnts, wstack, bstack)


def _prep_edges(ei):
    src = jnp.concatenate(
        [ei[0].astype(jnp.int32), jnp.zeros((EPAD - E,), jnp.int32)])
    dst = jnp.concatenate(
        [ei[1].astype(jnp.int32), jnp.full((EPAD - E,), N, jnp.int32)])
    return src, dst


def kernel(x, edge_index_r0, edge_index_r1, edge_index_r2,
           W0_r0, b0_r0, W0_r1, b0_r1, W0_r2, b0_r2,
           W1_r0, b1_r0, W1_r1, b1_r1, W1_r2, b1_r2):
    prepped = [_prep_edges(e)
               for e in (edge_index_r0, edge_index_r1, edge_index_r2)]
    src3 = jnp.stack([p[0] for p in prepped]).reshape(3, 16, CHUNKS, K)
    dst3 = jnp.stack([p[1] for p in prepped]).reshape(3, 16, CHUNKS, K)

    zb = jnp.zeros((K, DP), jnp.bfloat16)
    z16 = jnp.zeros((K, 16), jnp.float32)
    o16 = jnp.ones((K, 16), jnp.float32)

    xp = jnp.pad(x, ((0, NPAD - N), (0, 0))).astype(jnp.bfloat16)
    hparts0 = xp.reshape(NPAD, NP_PASS, DP).transpose(1, 0, 2)

    w0 = jnp.stack([W0_r0, W0_r1, W0_r2])
    b0 = jnp.stack([b0_r0, b0_r1, b0_r2])[:, None, :]
    w1 = jnp.stack([W1_r0, W1_r1, W1_r2])
    b1 = jnp.stack([b1_r0, b1_r1, b1_r2])[:, None, :]

    cnts = _sc_counts(dst3, o16, z16)
    parts0 = _sc_agg(hparts0, src3, dst3, zb)
    hparts1 = _tc_mm(parts0, cnts, w0, b0, emit_parts=True)
    parts1 = _sc_agg(hparts1, src3, dst3, zb)
    out = _tc_mm(parts1, cnts, w1, b1, emit_parts=False)
    return out[:N]


# speedup vs baseline: 4.0413x; 1.0458x over previous
import functools

import jax
import jax.numpy as jnp
from jax import lax
from jax.experimental import pallas as pl
from jax.experimental.pallas import tpu as pltpu
from jax.experimental.pallas import tpu_sc as plsc

N = 50000
D = 128
E = 200000

NPAD = 51200
EPAD = 200704
CHUNKS = 98
K = 128
BATCH = 14
NBATCH = CHUNKS // BATCH
RING = 6
BK = 1000
HROWS = N
ROWS_PER_TILE = NPAD // 16
ZCH = ROWS_PER_TILE // K
NP_PASS = 2
DP = D // NP_PASS

_mesh = plsc.VectorSubcoreMesh(core_axis_name="c", subcore_axis_name="s")


@functools.partial(
    pl.kernel,
    out_type=jax.ShapeDtypeStruct((3, NPAD, D), jnp.bfloat16),
    mesh=_mesh,
    scratch_types=(
        [pltpu.VMEM((BATCH, K), jnp.int32),
         pltpu.VMEM((BATCH, K), jnp.int32)]
        + [pltpu.VMEM((K, DP), jnp.bfloat16) for _ in range(RING)]
        + [pltpu.VMEM_SHARED((NPAD, DP), jnp.bfloat16)]
        + [pltpu.SemaphoreType.DMA] * (2 * RING + 1)
    ),
    compiler_params=pltpu.CompilerParams(use_tc_tiling_on_sc=False),
)
def _sc_agg(hp_hbm, src_hbm, dst_hbm, z_hbm, out_hbm, src_v, dst_v, *rest):
    stg = list(rest[:RING])
    acc = rest[RING]
    sgs = list(rest[RING + 1:2 * RING + 1])
    sss = list(rest[2 * RING + 1:3 * RING + 1])
    sz = rest[3 * RING + 1]
    c = lax.axis_index("c")
    s = lax.axis_index("s")
    hp_p = hp_hbm.at[c]
    for r in range(3):
        @pl.loop(0, ZCH)
        def _zero(j):
            pltpu.async_copy(
                z_hbm, acc.at[pl.ds(s * ROWS_PER_TILE + j * K, K)], sz)

        @pl.loop(0, ZCH)
        def _zero_wait(j):
            pltpu.make_async_copy(
                z_hbm, acc.at[pl.ds(s * ROWS_PER_TILE + j * K, K)],
                sz).wait()
        plsc.subcore_barrier()

        @pl.loop(0, NBATCH)
        def _batch(b):
            pltpu.sync_copy(src_hbm.at[r, s, pl.ds(b * BATCH, BATCH)], src_v)
            pltpu.sync_copy(dst_hbm.at[r, s, pl.ds(b * BATCH, BATCH)], dst_v)
            for q in range(RING):
                pltpu.async_copy(hp_p.at[src_v.at[q]], stg[q], sgs[q])
            for t in range(BATCH):
                k = t % RING
                if t >= 1 and (t - 1) + RING < BATCH:
                    k2 = (t - 1) % RING
                    pltpu.make_async_copy(
                        stg[k2], acc.at[dst_v.at[t - 1]], sss[k2]).wait()
                    pltpu.async_copy(
                        hp_p.at[src_v.at[t - 1 + RING]], stg[k2], sgs[k2])
                pltpu.make_async_copy(
                    hp_p.at[src_v.at[t]], stg[k], sgs[k]).wait()
                pltpu.async_copy(
                    stg[k], acc.at[dst_v.at[t]], sss[k], add=True)
            for t in range(BATCH - RING, BATCH):
                k = t % RING
                pltpu.make_async_copy(
                    stg[k], acc.at[dst_v.at[t]], sss[k]).wait()
        plsc.subcore_barrier()

        pltpu.sync_copy(
            acc.at[pl.ds(s * ROWS_PER_TILE, ROWS_PER_TILE)],
            out_hbm.at[r,
                       pl.ds(s * ROWS_PER_TILE, ROWS_PER_TILE),
                       pl.ds(c * DP, DP)])


@functools.partial(
    pl.kernel,
    out_type=jax.ShapeDtypeStruct((3, 2, NPAD, 16), jnp.float32),
    mesh=_mesh,
    scratch_types=[
        pltpu.VMEM((CHUNKS, K), jnp.int32),
        pltpu.VMEM((K, 16), jnp.float32),
        pltpu.VMEM((K, 16), jnp.float32),
        pltpu.VMEM_SHARED((NPAD, 16), jnp.float32),
        pltpu.SemaphoreType.DMA,
    ],
    compiler_params=pltpu.CompilerParams(use_tc_tiling_on_sc=False),
)
def _sc_counts(dst_hbm, ones_hbm, z_hbm, out_hbm,
               dst_v, ones_v, zero_v, acc, sem):
    c = lax.axis_index("c")
    s = lax.axis_index("s")
    pltpu.sync_copy(ones_hbm, ones_v)
    pltpu.sync_copy(z_hbm, zero_v)
    for r in range(3):
        pltpu.sync_copy(dst_hbm.at[r, s], dst_v)

        @pl.loop(0, ZCH)
        def _zero(j):
            pltpu.sync_copy(
                zero_v, acc.at[pl.ds(s * ROWS_PER_TILE + j * K, K)])
        plsc.subcore_barrier()

        @pl.loop(0, CHUNKS)
        def _edges(j):
            pltpu.async_copy(ones_v, acc.at[dst_v.at[j]], sem, add=True)

        @pl.loop(0, CHUNKS)
        def _edges_wait(j):
            pltpu.make_async_copy(
                ones_v, acc.at[dst_v.at[j]], sem).wait()
        plsc.subcore_barrier()

        pltpu.sync_copy(
            acc.at[pl.ds(s * ROWS_PER_TILE, ROWS_PER_TILE)],
            out_hbm.at[r, c, pl.ds(s * ROWS_PER_TILE, ROWS_PER_TILE)])


def _tc_body(parts_ref, cnt_ref, w_ref, b_ref, out_ref, *, emit_parts):
    acc = jnp.zeros((BK, D), jnp.float32)
    for r in range(3):
        cnt = cnt_ref[r].astype(jnp.float32)
        inv = 1.0 / jnp.maximum(cnt, 1.0)
        agg = parts_ref[r].astype(jnp.float32) * inv
        acc = acc + jnp.dot(agg, w_ref[r],
                            preferred_element_type=jnp.float32)
        acc = acc + jnp.where(cnt > 0.0, 1.0, 0.0) * b_ref[r]
    y = jnp.where(acc >= 0.0, acc, 0.01 * acc)
    if emit_parts:
        for p in range(NP_PASS):
            out_ref[p] = y[:, p * DP:(p + 1) * DP].astype(jnp.bfloat16)
    else:
        out_ref[...] = y


def _tc_mm(parts, cnt_rep, wstack, bstack, emit_parts):
    grid = (HROWS // BK,)
    if emit_parts:
        out_shape = jax.ShapeDtypeStruct((NP_PASS, HROWS, DP), jnp.bfloat16)
        out_spec = pl.BlockSpec((NP_PASS, BK, DP), lambda i: (0, i, 0))
    else:
        out_shape = jax.ShapeDtypeStruct((HROWS, D), jnp.float32)
        out_spec = pl.BlockSpec((BK, D), lambda i: (i, 0))
    return pl.pallas_call(
        functools.partial(_tc_body, emit_parts=emit_parts),
        grid=grid,
        in_specs=[
            pl.BlockSpec((3, BK, D), lambda i: (0, i, 0)),
            pl.BlockSpec((3, BK, D), lambda i: (0, i, 0)),
            pl.BlockSpec((3, D, D), lambda i: (0, 0, 0)),
            pl.BlockSpec((3, 1, D), lambda i: (0, 0, 0)),
        ],
        out_specs=out_spec,
        out_shape=out_shape,
    )(parts, cnt_rep, wstack, bstack)


def _prep_edges(ei):
    src = jnp.concatenate(
        [ei[0].astype(jnp.int32), jnp.zeros((EPAD - E,), jnp.int32)])
    dst = jnp.concatenate(
        [ei[1].astype(jnp.int32), jnp.full((EPAD - E,), N, jnp.int32)])
    return src, dst


def kernel(x, edge_index_r0, edge_index_r1, edge_index_r2,
           W0_r0, b0_r0, W0_r1, b0_r1, W0_r2, b0_r2,
           W1_r0, b1_r0, W1_r1, b1_r1, W1_r2, b1_r2):
    prepped = [_prep_edges(e)
               for e in (edge_index_r0, edge_index_r1, edge_index_r2)]
    src3 = jnp.stack([p[0] for p in prepped]).reshape(3, 16, CHUNKS, K)
    dst3 = jnp.stack([p[1] for p in prepped]).reshape(3, 16, CHUNKS, K)

    zb = jnp.zeros((K, DP), jnp.bfloat16)
    z16 = jnp.zeros((K, 16), jnp.float32)
    o16 = jnp.ones((K, 16), jnp.float32)

    hparts0 = (x.astype(jnp.bfloat16)
               .reshape(HROWS, NP_PASS, DP).transpose(1, 0, 2))

    w0 = jnp.stack([W0_r0, W0_r1, W0_r2])
    b0 = jnp.stack([b0_r0, b0_r1, b0_r2])[:, None, :]
    w1 = jnp.stack([W1_r0, W1_r1, W1_r2])
    b1 = jnp.stack([b1_r0, b1_r1, b1_r2])[:, None, :]

    cnts = _sc_counts(dst3, o16, z16)
    cnt_rep = jnp.broadcast_to(
        cnts[:, 0, :HROWS, 0:1].astype(jnp.bfloat16), (3, HROWS, D))

    parts0 = _sc_agg(hparts0, src3, dst3, zb)
    hparts1 = _tc_mm(parts0, cnt_rep, w0, b0, emit_parts=True)
    parts1 = _sc_agg(hparts1, src3, dst3, zb)
    return _tc_mm(parts1, cnt_rep, w1, b1, emit_parts=False)


# speedup vs baseline: 4.0923x; 1.0126x over previous
import functools

import jax
import jax.numpy as jnp
from jax import lax
from jax.experimental import pallas as pl
from jax.experimental.pallas import tpu as pltpu
from jax.experimental.pallas import tpu_sc as plsc

N = 50000
D = 128
E = 200000

NPAD = 51200
EPAD = 200704
CHUNKS = 98
K = 128
BATCH = 14
NBATCH = CHUNKS // BATCH
RING = 6
BK = 1024
HROWS = NPAD
ROWS_PER_TILE = NPAD // 16
ZCH = ROWS_PER_TILE // K
NP_PASS = 2
DP = D // NP_PASS

_mesh = plsc.VectorSubcoreMesh(core_axis_name="c", subcore_axis_name="s")


@functools.partial(
    pl.kernel,
    out_type=jax.ShapeDtypeStruct((3, NPAD, D), jnp.bfloat16),
    mesh=_mesh,
    scratch_types=(
        [pltpu.VMEM((BATCH, K), jnp.int32),
         pltpu.VMEM((BATCH, K), jnp.int32)]
        + [pltpu.VMEM((K, DP), jnp.bfloat16) for _ in range(RING)]
        + [pltpu.VMEM_SHARED((NPAD, DP), jnp.bfloat16)]
        + [pltpu.SemaphoreType.DMA] * (2 * RING + 1)
    ),
    compiler_params=pltpu.CompilerParams(use_tc_tiling_on_sc=False),
)
def _sc_agg(hp_hbm, src_hbm, dst_hbm, z_hbm, out_hbm, src_v, dst_v, *rest):
    stg = list(rest[:RING])
    acc = rest[RING]
    sgs = list(rest[RING + 1:2 * RING + 1])
    sss = list(rest[2 * RING + 1:3 * RING + 1])
    sz = rest[3 * RING + 1]
    c = lax.axis_index("c")
    s = lax.axis_index("s")
    hp_p = hp_hbm.at[c]
    for r in range(3):
        @pl.loop(0, ZCH)
        def _zero(j):
            pltpu.async_copy(
                z_hbm, acc.at[pl.ds(s * ROWS_PER_TILE + j * K, K)], sz)

        @pl.loop(0, ZCH)
        def _zero_wait(j):
            pltpu.make_async_copy(
                z_hbm, acc.at[pl.ds(s * ROWS_PER_TILE + j * K, K)],
                sz).wait()
        plsc.subcore_barrier()

        @pl.loop(0, NBATCH)
        def _batch(b):
            pltpu.sync_copy(src_hbm.at[r, s, pl.ds(b * BATCH, BATCH)], src_v)
            pltpu.sync_copy(dst_hbm.at[r, s, pl.ds(b * BATCH, BATCH)], dst_v)
            for q in range(RING):
                pltpu.async_copy(hp_p.at[src_v.at[q]], stg[q], sgs[q])
            for t in range(BATCH):
                k = t % RING
                if t >= 1 and (t - 1) + RING < BATCH:
                    k2 = (t - 1) % RING
                    pltpu.make_async_copy(
                        stg[k2], acc.at[dst_v.at[t - 1]], sss[k2]).wait()
                    pltpu.async_copy(
                        hp_p.at[src_v.at[t - 1 + RING]], stg[k2], sgs[k2])
                pltpu.make_async_copy(
                    hp_p.at[src_v.at[t]], stg[k], sgs[k]).wait()
                pltpu.async_copy(
                    stg[k], acc.at[dst_v.at[t]], sss[k], add=True)
            for t in range(BATCH - RING, BATCH):
                k = t % RING
                pltpu.make_async_copy(
                    stg[k], acc.at[dst_v.at[t]], sss[k]).wait()
        plsc.subcore_barrier()

        pltpu.sync_copy(
            acc.at[pl.ds(s * ROWS_PER_TILE, ROWS_PER_TILE)],
            out_hbm.at[r,
                       pl.ds(s * ROWS_PER_TILE, ROWS_PER_TILE),
                       pl.ds(c * DP, DP)])


@functools.partial(
    pl.kernel,
    out_type=jax.ShapeDtypeStruct((3, 2, NPAD, 16), jnp.float32),
    mesh=_mesh,
    scratch_types=[
        pltpu.VMEM((CHUNKS, K), jnp.int32),
        pltpu.VMEM((K, 16), jnp.float32),
        pltpu.VMEM((K, 16), jnp.float32),
        pltpu.VMEM_SHARED((NPAD, 16), jnp.float32),
        pltpu.SemaphoreType.DMA,
    ],
    compiler_params=pltpu.CompilerParams(use_tc_tiling_on_sc=False),
)
def _sc_counts(dst_hbm, ones_hbm, z_hbm, out_hbm,
               dst_v, ones_v, zero_v, acc, sem):
    c = lax.axis_index("c")
    s = lax.axis_index("s")
    pltpu.sync_copy(ones_hbm, ones_v)
    pltpu.sync_copy(z_hbm, zero_v)
    for r in range(3):
        pltpu.sync_copy(dst_hbm.at[r, s], dst_v)

        @pl.loop(0, ZCH)
        def _zero(j):
            pltpu.sync_copy(
                zero_v, acc.at[pl.ds(s * ROWS_PER_TILE + j * K, K)])
        plsc.subcore_barrier()

        @pl.loop(0, CHUNKS)
        def _edges(j):
            pltpu.async_copy(ones_v, acc.at[dst_v.at[j]], sem, add=True)

        @pl.loop(0, CHUNKS)
        def _edges_wait(j):
            pltpu.make_async_copy(
                ones_v, acc.at[dst_v.at[j]], sem).wait()
        plsc.subcore_barrier()

        pltpu.sync_copy(
            acc.at[pl.ds(s * ROWS_PER_TILE, ROWS_PER_TILE)],
            out_hbm.at[r, c, pl.ds(s * ROWS_PER_TILE, ROWS_PER_TILE)])


def _tc_body(parts_ref, cnt_ref, w_ref, b_ref, out_ref, *, emit_parts):
    acc = jnp.zeros((BK, D), jnp.float32)
    for r in range(3):
        cnt = cnt_ref[r].astype(jnp.float32)
        inv = 1.0 / jnp.maximum(cnt, 1.0)
        agg = parts_ref[r].astype(jnp.float32) * inv
        acc = acc + jnp.dot(agg, w_ref[r],
                            preferred_element_type=jnp.float32)
        acc = acc + jnp.where(cnt > 0.0, 1.0, 0.0) * b_ref[r]
    y = jnp.where(acc >= 0.0, acc, 0.01 * acc)
    if emit_parts:
        for p in range(NP_PASS):
            out_ref[p] = y[:, p * DP:(p + 1) * DP].astype(jnp.bfloat16)
    else:
        out_ref[...] = y


def _tc_mm(parts, cnt_rep, wstack, bstack, emit_parts):
    grid = (HROWS // BK,)
    if emit_parts:
        out_shape = jax.ShapeDtypeStruct((NP_PASS, HROWS, DP), jnp.bfloat16)
        out_spec = pl.BlockSpec((NP_PASS, BK, DP), lambda i: (0, i, 0))
    else:
        out_shape = jax.ShapeDtypeStruct((HROWS, D), jnp.float32)
        out_spec = pl.BlockSpec((BK, D), lambda i: (i, 0))
    return pl.pallas_call(
        functools.partial(_tc_body, emit_parts=emit_parts),
        grid=grid,
        in_specs=[
            pl.BlockSpec((3, BK, D), lambda i: (0, i, 0)),
            pl.BlockSpec((3, BK, D), lambda i: (0, i, 0)),
            pl.BlockSpec((3, D, D), lambda i: (0, 0, 0)),
            pl.BlockSpec((3, 1, D), lambda i: (0, 0, 0)),
        ],
        out_specs=out_spec,
        out_shape=out_shape,
    )(parts, cnt_rep, wstack, bstack)


def _prep_edges(ei):
    src = jnp.concatenate(
        [ei[0].astype(jnp.int32), jnp.zeros((EPAD - E,), jnp.int32)])
    dst = jnp.concatenate(
        [ei[1].astype(jnp.int32), jnp.full((EPAD - E,), N, jnp.int32)])
    return src, dst


def kernel(x, edge_index_r0, edge_index_r1, edge_index_r2,
           W0_r0, b0_r0, W0_r1, b0_r1, W0_r2, b0_r2,
           W1_r0, b1_r0, W1_r1, b1_r1, W1_r2, b1_r2):
    prepped = [_prep_edges(e)
               for e in (edge_index_r0, edge_index_r1, edge_index_r2)]
    src3 = jnp.stack([p[0] for p in prepped]).reshape(3, 16, CHUNKS, K)
    dst3 = jnp.stack([p[1] for p in prepped]).reshape(3, 16, CHUNKS, K)

    zb = jnp.zeros((K, DP), jnp.bfloat16)
    z16 = jnp.zeros((K, 16), jnp.float32)
    o16 = jnp.ones((K, 16), jnp.float32)

    xp = jnp.pad(x, ((0, NPAD - N), (0, 0))).astype(jnp.bfloat16)
    hparts0 = xp.reshape(HROWS, NP_PASS, DP).transpose(1, 0, 2)

    w0 = jnp.stack([W0_r0, W0_r1, W0_r2])
    b0 = jnp.stack([b0_r0, b0_r1, b0_r2])[:, None, :]
    w1 = jnp.stack([W1_r0, W1_r1, W1_r2])
    b1 = jnp.stack([b1_r0, b1_r1, b1_r2])[:, None, :]

    cnts = _sc_counts(dst3, o16, z16)
    cnt_rep = jnp.broadcast_to(
        cnts[:, 0, :HROWS, 0:1].astype(jnp.bfloat16), (3, HROWS, D))
    cnt_rep = lax.optimization_barrier(cnt_rep)

    parts0 = _sc_agg(hparts0, src3, dst3, zb)
    hparts1 = _tc_mm(parts0, cnt_rep, w0, b0, emit_parts=True)
    parts1 = _sc_agg(hparts1, src3, dst3, zb)
    return _tc_mm(parts1, cnt_rep, w1, b1, emit_parts=False)[:N]


# speedup vs baseline: 4.1034x; 1.0027x over previous
import functools

import jax
import jax.numpy as jnp
from jax import lax
from jax.experimental import pallas as pl
from jax.experimental.pallas import tpu as pltpu
from jax.experimental.pallas import tpu_sc as plsc

N = 50000
D = 128
E = 200000

NPAD = 51200
EPAD = 200704
CHUNKS = 98
K = 128
BATCH = 14
NBATCH = CHUNKS // BATCH
RING = 6
BK = 1024
HROWS = NPAD
ROWS_PER_TILE = NPAD // 16
ZCH = ROWS_PER_TILE // K
NP_PASS = 2
DP = D // NP_PASS

_mesh = plsc.VectorSubcoreMesh(core_axis_name="c", subcore_axis_name="s")


@functools.partial(
    pl.kernel,
    out_type=jax.ShapeDtypeStruct((3, NPAD, D), jnp.bfloat16),
    mesh=_mesh,
    scratch_types=(
        [pltpu.VMEM((BATCH, K), jnp.int32),
         pltpu.VMEM((BATCH, K), jnp.int32)]
        + [pltpu.VMEM((K, DP), jnp.bfloat16) for _ in range(RING)]
        + [pltpu.VMEM_SHARED((NPAD, DP), jnp.bfloat16)]
        + [pltpu.SemaphoreType.DMA] * (2 * RING + 1)
    ),
    compiler_params=pltpu.CompilerParams(use_tc_tiling_on_sc=False),
)
def _sc_agg(hp_hbm, src_hbm, dst_hbm, z_hbm, out_hbm, src_v, dst_v, *rest):
    stg = list(rest[:RING])
    acc = rest[RING]
    sgs = list(rest[RING + 1:2 * RING + 1])
    sss = list(rest[2 * RING + 1:3 * RING + 1])
    sz = rest[3 * RING + 1]
    c = lax.axis_index("c")
    s = lax.axis_index("s")
    hp_p = hp_hbm.at[c]
    for r in range(3):
        @pl.loop(0, ZCH)
        def _zero(j):
            pltpu.async_copy(
                z_hbm, acc.at[pl.ds(s * ROWS_PER_TILE + j * K, K)], sz)

        @pl.loop(0, ZCH)
        def _zero_wait(j):
            pltpu.make_async_copy(
                z_hbm, acc.at[pl.ds(s * ROWS_PER_TILE + j * K, K)],
                sz).wait()
        plsc.subcore_barrier()

        @pl.loop(0, NBATCH)
        def _batch(b):
            pltpu.sync_copy(src_hbm.at[r, s, pl.ds(b * BATCH, BATCH)], src_v)
            pltpu.sync_copy(dst_hbm.at[r, s, pl.ds(b * BATCH, BATCH)], dst_v)
            for q in range(RING):
                pltpu.async_copy(hp_p.at[src_v.at[q]], stg[q], sgs[q])
            for t in range(BATCH):
                k = t % RING
                if t >= 1 and (t - 1) + RING < BATCH:
                    k2 = (t - 1) % RING
                    pltpu.make_async_copy(
                        stg[k2], acc.at[dst_v.at[t - 1]], sss[k2]).wait()
                    pltpu.async_copy(
                        hp_p.at[src_v.at[t - 1 + RING]], stg[k2], sgs[k2])
                pltpu.make_async_copy(
                    hp_p.at[src_v.at[t]], stg[k], sgs[k]).wait()
                pltpu.async_copy(
                    stg[k], acc.at[dst_v.at[t]], sss[k], add=True)
            for t in range(BATCH - RING, BATCH):
                k = t % RING
                pltpu.make_async_copy(
                    stg[k], acc.at[dst_v.at[t]], sss[k]).wait()
        plsc.subcore_barrier()

        pltpu.sync_copy(
            acc.at[pl.ds(s * ROWS_PER_TILE, ROWS_PER_TILE)],
            out_hbm.at[r,
                       pl.ds(s * ROWS_PER_TILE, ROWS_PER_TILE),
                       pl.ds(c * DP, DP)])


@functools.partial(
    pl.kernel,
    out_type=jax.ShapeDtypeStruct((3, 2, NPAD, 16), jnp.float32),
    mesh=_mesh,
    scratch_types=[
        pltpu.VMEM((CHUNKS, K), jnp.int32),
        pltpu.VMEM((K, 16), jnp.float32),
        pltpu.VMEM((K, 16), jnp.float32),
        pltpu.VMEM_SHARED((NPAD, 16), jnp.float32),
        pltpu.SemaphoreType.DMA,
    ],
    compiler_params=pltpu.CompilerParams(use_tc_tiling_on_sc=False),
)
def _sc_counts(dst_hbm, ones_hbm, z_hbm, out_hbm,
               dst_v, ones_v, zero_v, acc, sem):
    c = lax.axis_index("c")
    s = lax.axis_index("s")
    pltpu.sync_copy(ones_hbm, ones_v)
    pltpu.sync_copy(z_hbm, zero_v)
    for r in range(3):
        pltpu.sync_copy(dst_hbm.at[r, s], dst_v)

        @pl.loop(0, ZCH)
        def _zero(j):
            pltpu.async_copy(
                zero_v, acc.at[pl.ds(s * ROWS_PER_TILE + j * K, K)], sem)

        @pl.loop(0, ZCH)
        def _zero_wait(j):
            pltpu.make_async_copy(
                zero_v, acc.at[pl.ds(s * ROWS_PER_TILE + j * K, K)],
                sem).wait()
        plsc.subcore_barrier()

        @pl.loop(0, CHUNKS)
        def _edges(j):
            pltpu.async_copy(ones_v, acc.at[dst_v.at[j]], sem, add=True)

        @pl.loop(0, CHUNKS)
        def _edges_wait(j):
            pltpu.make_async_copy(
                ones_v, acc.at[dst_v.at[j]], sem).wait()
        plsc.subcore_barrier()

        pltpu.sync_copy(
            acc.at[pl.ds(s * ROWS_PER_TILE, ROWS_PER_TILE)],
            out_hbm.at[r, c, pl.ds(s * ROWS_PER_TILE, ROWS_PER_TILE)])


def _tc_body(parts_ref, cnt_ref, w_ref, b_ref, out_ref, *, emit_parts):
    acc = jnp.zeros((BK, D), jnp.float32)
    for r in range(3):
        cnt = cnt_ref[r].astype(jnp.float32)
        inv = 1.0 / jnp.maximum(cnt, 1.0)
        agg = parts_ref[r].astype(jnp.float32) * inv
        acc = acc + jnp.dot(agg, w_ref[r],
                            preferred_element_type=jnp.float32)
        acc = acc + jnp.where(cnt > 0.0, 1.0, 0.0) * b_ref[r]
    y = jnp.where(acc >= 0.0, acc, 0.01 * acc)
    if emit_parts:
        for p in range(NP_PASS):
            out_ref[p] = y[:, p * DP:(p + 1) * DP].astype(jnp.bfloat16)
    else:
        out_ref[...] = y


def _tc_mm(parts, cnt_rep, wstack, bstack, emit_parts):
    grid = (HROWS // BK,)
    if emit_parts:
        out_shape = jax.ShapeDtypeStruct((NP_PASS, HROWS, DP), jnp.bfloat16)
        out_spec = pl.BlockSpec((NP_PASS, BK, DP), lambda i: (0, i, 0))
    else:
        out_shape = jax.ShapeDtypeStruct((HROWS, D), jnp.float32)
        out_spec = pl.BlockSpec((BK, D), lambda i: (i, 0))
    return pl.pallas_call(
        functools.partial(_tc_body, emit_parts=emit_parts),
        grid=grid,
        in_specs=[
            pl.BlockSpec((3, BK, D), lambda i: (0, i, 0)),
            pl.BlockSpec((3, BK, D), lambda i: (0, i, 0)),
            pl.BlockSpec((3, D, D), lambda i: (0, 0, 0)),
            pl.BlockSpec((3, 1, D), lambda i: (0, 0, 0)),
        ],
        out_specs=out_spec,
        out_shape=out_shape,
    )(parts, cnt_rep, wstack, bstack)


def _prep_edges(ei):
    src = jnp.concatenate(
        [ei[0].astype(jnp.int32), jnp.zeros((EPAD - E,), jnp.int32)])
    dst = jnp.concatenate(
        [ei[1].astype(jnp.int32), jnp.full((EPAD - E,), N, jnp.int32)])
    return src, dst


def kernel(x, edge_index_r0, edge_index_r1, edge_index_r2,
           W0_r0, b0_r0, W0_r1, b0_r1, W0_r2, b0_r2,
           W1_r0, b1_r0, W1_r1, b1_r1, W1_r2, b1_r2):
    prepped = [_prep_edges(e)
               for e in (edge_index_r0, edge_index_r1, edge_index_r2)]
    src3 = jnp.stack([p[0] for p in prepped]).reshape(3, 16, CHUNKS, K)
    dst3 = jnp.stack([p[1] for p in prepped]).reshape(3, 16, CHUNKS, K)

    zb = jnp.zeros((K, DP), jnp.bfloat16)
    z16 = jnp.zeros((K, 16), jnp.float32)
    o16 = jnp.ones((K, 16), jnp.float32)

    cnts = _sc_counts(dst3, o16, z16)

    xp = jnp.pad(x, ((0, NPAD - N), (0, 0))).astype(jnp.bfloat16)
    hparts0 = xp.reshape(HROWS, NP_PASS, DP).transpose(1, 0, 2)

    w0 = jnp.stack([W0_r0, W0_r1, W0_r2])
    b0 = jnp.stack([b0_r0, b0_r1, b0_r2])[:, None, :]
    w1 = jnp.stack([W1_r0, W1_r1, W1_r2])
    b1 = jnp.stack([b1_r0, b1_r1, b1_r2])[:, None, :]
    cnt_rep = jnp.broadcast_to(
        cnts[:, 0, :HROWS, 0:1].astype(jnp.bfloat16), (3, HROWS, D))
    cnt_rep = lax.optimization_barrier(cnt_rep)

    parts0 = _sc_agg(hparts0, src3, dst3, zb)
    hparts1 = _tc_mm(parts0, cnt_rep, w0, b0, emit_parts=True)
    parts1 = _sc_agg(hparts1, src3, dst3, zb)
    return _tc_mm(parts1, cnt_rep, w1, b1, emit_parts=False)[:N]
